# baseline (speedup 1.0000x reference)
import numpy as np
import jax
import jax.numpy as jnp
from jax.experimental import pallas as pl
from jax.experimental.pallas import tpu as pltpu

_IDX = [162, 1098, 1377]
BLOCK_ROWS = 1024


def _copyfix_kernel(img_hbm, out_ref, sem):
    i = pl.program_id(0)
    pltpu.make_async_copy(
        img_hbm.at[pl.ds(i * BLOCK_ROWS, BLOCK_ROWS), :],
        out_ref,
        sem,
    ).start()
    pltpu.make_async_copy(
        img_hbm.at[pl.ds(i * BLOCK_ROWS, BLOCK_ROWS), :],
        out_ref,
        sem,
    ).wait()
    for c in _IDX:
        out_ref[:, c : c + 1] = jnp.zeros((BLOCK_ROWS, 1), jnp.float32)


def kernel(img):
    n_rows, n_cols = img.shape
    grid = (n_rows // BLOCK_ROWS,)
    return pl.pallas_call(
        _copyfix_kernel,
        grid=grid,
        in_specs=[pl.BlockSpec(memory_space=pl.ANY)],
        out_specs=pl.BlockSpec((BLOCK_ROWS, n_cols), lambda i: (i, 0)),
        out_shape=jax.ShapeDtypeStruct((n_rows, n_cols), img.dtype),
        scratch_shapes=[pltpu.SemaphoreType.DMA],
    )(img)
